# Initial kernel scaffold; baseline (speedup 1.0000x reference)
import jax
import jax.numpy as jnp
from jax.experimental import pallas as pl


def kernel(src, table, W, b):
    raise NotImplementedError("write your pallas kernel here")


# speedup vs baseline: 1.2620x; 1.2620x over previous
import jax
import jax.numpy as jnp
from jax import lax
from jax.experimental import pallas as pl
from jax.experimental.pallas import tpu as pltpu
from jax.experimental.pallas import tpu_sc as plsc

_NC = 2
_NS = 16
_NW = _NC * _NS
_CH = 128
_D = 64


def _fused_table_body(t_ref, wt_ref, b_ref, o_ref):
    o_ref[...] = (
        jnp.dot(t_ref[...], wt_ref[...], preferred_element_type=jnp.float32)
        + b_ref[...]
    )


def _gather_body(fused_hbm, idx_hbm, out_hbm, idx_v, rows_v, sem):
    wid = lax.axis_index("s") * _NC + lax.axis_index("c")
    nch = idx_hbm.shape[0] // _NW
    row0 = wid * nch
    pltpu.sync_copy(idx_hbm.at[pl.ds(row0, nch)], idx_v)

    def step(j, carry):
        pltpu.async_copy(fused_hbm.at[idx_v.at[j]], rows_v, sem).wait()
        pltpu.sync_copy(rows_v, out_hbm.at[pl.ds((row0 + j) * _CH, _CH)])
        return carry

    lax.fori_loop(0, nch, step, 0)


def kernel(src, table, W, b):
    B, S = src.shape
    total = B * S
    tpad = jnp.pad(table, ((0, (-table.shape[0]) % 8), (0, 0)))
    fused = pl.pallas_call(
        _fused_table_body,
        out_shape=jax.ShapeDtypeStruct((tpad.shape[0], _D), jnp.float32),
    )(tpad, W.T, b[None, :])

    idx = src.reshape(total // _CH, _CH)
    nch = idx.shape[0] // _NW
    out = pl.kernel(
        _gather_body,
        out_type=jax.ShapeDtypeStruct((total, _D), jnp.float32),
        mesh=plsc.VectorSubcoreMesh(core_axis_name="c", subcore_axis_name="s"),
        compiler_params=pltpu.CompilerParams(use_tc_tiling_on_sc=False),
        scratch_types=[
            pltpu.VMEM((nch, _CH), jnp.int32),
            pltpu.VMEM((_CH, _D), jnp.float32),
            pltpu.SemaphoreType.DMA,
        ],
    )(fused, idx)
    return out.reshape(B, S, _D)


# speedup vs baseline: 4.8132x; 3.8140x over previous
import jax
import jax.numpy as jnp
from jax import lax
from jax.experimental import pallas as pl
from jax.experimental.pallas import tpu as pltpu
from jax.experimental.pallas import tpu_sc as plsc

_NC = 2
_NS = 16
_NW = _NC * _NS
_CH = 128
_G = 5
_NBUF = 2
_D = 64


def _fused_table_body(t_ref, wt_ref, b_ref, o_ref):
    o_ref[...] = (
        jnp.dot(t_ref[...], wt_ref[...], preferred_element_type=jnp.float32)
        + b_ref[...]
    )


def _gather_body(fused_hbm, idx_hbm, out_hbm, fused_s, idx_v, big,
                 gsem0, gsem1, ssem0, ssem1):
    gsem = (gsem0, gsem1)
    ssem = (ssem0, ssem1)
    wid = lax.axis_index("s") * _NC + lax.axis_index("c")
    nch = idx_hbm.shape[0] // _NW
    ngroups = nch // _G
    rows_per_group = _G * _CH
    row0 = wid * nch * _CH

    @pl.when(lax.axis_index("s") == 0)
    def _():
        pltpu.sync_copy(fused_hbm, fused_s)

    plsc.subcore_barrier()
    pltpu.sync_copy(idx_hbm.at[pl.ds(wid * nch, nch)], idx_v)

    def fire_gathers(g, b):
        for k in range(_G):
            pltpu.async_copy(
                fused_s.at[idx_v.at[g * _G + k]],
                big.at[b, pl.ds(k * _CH, _CH)],
                gsem[b],
            )

    def drain(src, dst, sem):
        pltpu.make_async_copy(src, dst, sem).wait()

    fire_gathers(0, 0)

    def step(i, carry):
        for b in range(_NBUF):
            g = i * _NBUF + b
            drain(out_hbm.at[pl.ds(0, rows_per_group)], big.at[b], gsem[b])
            pltpu.async_copy(
                big.at[b],
                out_hbm.at[pl.ds(row0 + g * rows_per_group, rows_per_group)],
                ssem[b],
            )
            b2 = (b + 1) % _NBUF

            @pl.when(g + 1 < ngroups)
            def _():
                @pl.when(g >= 1)
                def _():
                    drain(big.at[b2],
                          out_hbm.at[pl.ds(0, rows_per_group)], ssem[b2])

                fire_gathers(g + 1, b2)

        return carry

    lax.fori_loop(0, ngroups // _NBUF, step, 0)
    drain(big.at[0], out_hbm.at[pl.ds(0, rows_per_group)], ssem[0])
    drain(big.at[1], out_hbm.at[pl.ds(0, rows_per_group)], ssem[1])


def kernel(src, table, W, b):
    B, S = src.shape
    total = B * S
    tpad = jnp.pad(table, ((0, (-table.shape[0]) % 8), (0, 0)))
    fused = pl.pallas_call(
        _fused_table_body,
        out_shape=jax.ShapeDtypeStruct((tpad.shape[0], _D), jnp.float32),
    )(tpad, W.T, b[None, :])

    idx = src.reshape(total // _CH, _CH)
    nch = idx.shape[0] // _NW
    out = pl.kernel(
        _gather_body,
        out_type=jax.ShapeDtypeStruct((total, _D), jnp.float32),
        mesh=plsc.VectorSubcoreMesh(core_axis_name="c", subcore_axis_name="s"),
        compiler_params=pltpu.CompilerParams(use_tc_tiling_on_sc=False),
        scratch_types=[
            pltpu.VMEM_SHARED(fused.shape, jnp.float32),
            pltpu.VMEM((nch, _CH), jnp.int32),
            pltpu.VMEM((_NBUF, _G * _CH, _D), jnp.float32),
            pltpu.SemaphoreType.DMA,
            pltpu.SemaphoreType.DMA,
            pltpu.SemaphoreType.DMA,
            pltpu.SemaphoreType.DMA,
        ],
    )(fused, idx)
    return out.reshape(B, S, _D)
